# baseline (speedup 1.0000x reference)
import functools
import math

import jax
import jax.numpy as jnp
from jax import lax
from jax.experimental import pallas as pl
from jax.experimental.pallas import tpu as pltpu
from jax.experimental.pallas import tpu_sc as plsc

N = 32 * 1 * 512 * 512
N_MIN = N // 16
THRESH = -math.log(0.7)

NC = 2
NS = 16
NW = NC * NS
L = 16
IMG = 512
CH = 16384
CHR = CH // IMG
NCH = IMG * IMG // CH
NBUF = 2

NB = 1024
RS = NB + 8
UNROLL = 4
SCALE = (NB - 1) / THRESH

_C0 = 3.50755203726294e-06
_C1 = 0.9997924566268921
_C2 = -0.49697792530059814
_C3 = 0.31459054350852966
_C4 = -0.1887826770544052
_C5 = 0.0817268118262291
_C6 = -0.01720806024968624


def _sc_body(x_hbm, y_hbm, z_hbm, sum_out, xb, yb, hsum, sx0, sx1, sy0, sy1):
    wid = lax.axis_index("s") * NC + lax.axis_index("c")
    sems = ((sx0, sy0), (sx1, sy1))

    pltpu.sync_copy(z_hbm, hsum)

    lane_off = lax.iota(jnp.int32, 16) * RS

    def _start(c, b):
        sx, sy = sems[b]
        rows = pl.ds(c * CHR, CHR)
        hx = pltpu.async_copy(x_hbm.at[wid, rows], xb.at[b], sx)
        hy = pltpu.async_copy(y_hbm.at[wid, rows], yb.at[b], sy)
        return hx, hy

    def _process(b, cnt_in):
        def ibody(i, cnt_acc):
            r = i >> 5
            col = (i & 31) * L
            x = xb[b, r, pl.ds(col, L)]
            y = yb[b, r, pl.ds(col, L)]
            u = jnp.exp(-jnp.abs(x))
            u2 = u * u
            u4 = u2 * u2
            pa = _C0 + _C1 * u
            pb = _C2 + _C3 * u
            pc = (_C4 + _C5 * u) + u2 * _C6
            p = (pa + u2 * pb) + u4 * pc
            loss = jnp.maximum(x, 0.0) - x * y + p
            binf = loss * SCALE
            cnt_acc = cnt_acc + jnp.where(binf >= float(NB - 1), 1.0, 0.0)
            binf = jnp.minimum(binf, float(NB - 1))
            idx = lane_off + binf.astype(jnp.int32)
            plsc.addupdate_scatter(hsum, [idx], loss)
            return cnt_acc

        return plsc.parallel_loop(
            0, CH // L, unroll=UNROLL, carry=cnt_in)(ibody)

    cnt_acc = jnp.zeros((L,), jnp.float32)
    handles = [None] * NCH
    for c in range(NBUF):
        handles[c] = _start(c, c % NBUF)
    for c in range(NCH):
        b = c % NBUF
        hx, hy = handles[c]
        hx.wait()
        hy.wait()
        cnt_acc = _process(b, cnt_acc)
        if c + NBUF < NCH:
            handles[c + NBUF] = _start(c + NBUF, b)

    plsc.addupdate_scatter(hsum, [lane_off + NB], cnt_acc)
    pltpu.sync_copy(hsum, sum_out.at[wid])


@functools.cache
def _sc_hist():
    return pl.kernel(
        _sc_body,
        out_type=jax.ShapeDtypeStruct((NW, L * RS), jnp.float32),
        mesh=plsc.VectorSubcoreMesh(core_axis_name="c", subcore_axis_name="s"),
        compiler_params=pltpu.CompilerParams(needs_layout_passes=False),
        scratch_types=[
            pltpu.VMEM((NBUF, CHR, IMG), jnp.float32),
            pltpu.VMEM((NBUF, CHR, IMG), jnp.float32),
            pltpu.VMEM((L * RS,), jnp.float32),
            pltpu.SemaphoreType.DMA,
            pltpu.SemaphoreType.DMA,
            pltpu.SemaphoreType.DMA,
            pltpu.SemaphoreType.DMA,
        ],
    )


def _finalize_body(sum_ref, o_ref):
    sums = sum_ref[...]
    sum_b = jnp.sum(sums, axis=0, keepdims=True)

    col = lax.broadcasted_iota(jnp.int32, (1, RS), 1)
    cnt_gt = jnp.sum(jnp.where(col == NB, sum_b, 0.0))
    sum_gt = jnp.sum(jnp.where(col == NB - 1, sum_b, 0.0))
    masked_mean = sum_gt / jnp.maximum(cnt_gt, 1.0)

    below = col <= NB - 2
    centers = (col.astype(jnp.float32) + 0.5) * (1.0 / SCALE)
    cnt_e = jnp.where(below, sum_b / centers, 0.0)
    sum_e = jnp.where(below, sum_b, 0.0)

    ii = lax.broadcasted_iota(jnp.int32, (RS, RS), 0)
    jj = lax.broadcasted_iota(jnp.int32, (RS, RS), 1)
    tri = (ii > jj).astype(jnp.float32)
    stacked = jnp.concatenate([cnt_e, sum_e], axis=0)
    above = jax.lax.dot_general(
        stacked, tri, (((1,), (0,)), ((), ())),
        precision=jax.lax.Precision.HIGHEST,
        preferred_element_type=jnp.float32,
    )
    n_min = float(N_MIN)
    ca = above[0:1, :] + cnt_gt
    sa = above[1:2, :] + sum_gt

    is_cut = below & (ca < n_min) & (ca + cnt_e >= n_min)
    avg_b = sum_e / jnp.maximum(cnt_e, 1e-30)
    topk_sum = jnp.sum(jnp.where(is_cut, sa + (n_min - ca) * avg_b, 0.0))
    topk_mean = topk_sum / n_min

    res = jnp.where(cnt_gt < n_min, topk_mean, masked_mean)
    o_ref[...] = jnp.broadcast_to(res, (1, 1))


def kernel(logits, labels):
    xf = logits[:, 0]
    yf = labels[:, 0]
    zeros = jnp.zeros((L * RS,), jnp.float32)
    sum_p = _sc_hist()(xf, yf, zeros)
    sum2 = sum_p.reshape(NW * L, RS)
    out = pl.pallas_call(
        _finalize_body,
        out_shape=jax.ShapeDtypeStruct((1, 1), jnp.float32),
    )(sum2)
    return out[0, 0]
